# Initial kernel scaffold; baseline (speedup 1.0000x reference)
import jax
import jax.numpy as jnp
from jax.experimental import pallas as pl


def kernel(mu, pheromone, Wq, bq, Wk, bk, Wv, bv, W1, b1, W2, b2, Wo, bo):
    raise NotImplementedError("write your pallas kernel here")


# speedup vs baseline: 7.1260x; 7.1260x over previous
import functools
import math

import jax
import jax.numpy as jnp
from jax.experimental import pallas as pl

_WINDOW, _K, _ALPHA = 4, 8, 0.3
_F32 = jnp.float32


def _proj_kernel(mu_ref, w_ref, b_ref, q_ref, k_ref, v_ref, s_ref, t_ref):
    d = mu_ref.shape[-1]
    proj = jax.lax.dot_general(
        mu_ref[...], w_ref[...], (((1,), (0,)), ((), ())),
        preferred_element_type=_F32)
    proj = proj + b_ref[...]
    q_ref[...] = proj[:, 0 * d:1 * d]
    k_ref[...] = proj[:, 1 * d:2 * d]
    v_ref[...] = proj[:, 2 * d:3 * d]
    s_ref[...] = proj[:, 3 * d:4 * d]
    t_ref[...] = proj[:, 4 * d:5 * d]


def _local_kernel(s_ref, tc_ref, tp_ref, w2t_ref, woat_ref, b1_ref, bvec_ref,
                  out_ref):
    i = pl.program_id(1)
    rb = s_ref.shape[1]
    s = s_ref[0]
    tc = tc_ref[0]
    live = (i > 0).astype(_F32)
    tail = tp_ref[0][rb - _WINDOW:, :] * live
    text = jnp.concatenate([tail, tc], axis=0)
    b1 = b1_ref[...]
    acc = jnp.zeros_like(s)
    for w in range(1, _WINDOW + 1):
        x = s + text[_WINDOW - w:_WINDOW - w + rb, :] + b1
        acc = acc + x * jax.nn.sigmoid(x)
    h = acc * (1.0 / _WINDOW)
    local = jax.lax.dot_general(h, w2t_ref[...], (((1,), (0,)), ((), ())),
                                preferred_element_type=_F32)
    out_ref[0] = jax.lax.dot_general(
        local, woat_ref[...], (((1,), (0,)), ((), ())),
        preferred_element_type=_F32) + bvec_ref[...]


def _attn_kernel(q_ref, k_ref, v_ref, ph_ref, part_ref, wobt_ref, out_ref):
    i = pl.program_id(1)
    rb, d = q_ref.shape[1], q_ref.shape[2]
    n = k_ref.shape[1]
    q = q_ref[0]
    scores = jax.lax.dot_general(
        q, k_ref[0], (((1,), (1,)), ((), ())),
        preferred_element_type=_F32) * (1.0 / math.sqrt(d))
    scores = scores + _ALPHA * ph_ref[0]
    rows = i * rb + jax.lax.broadcasted_iota(jnp.int32, (rb, n), 0)
    cols = jax.lax.broadcasted_iota(jnp.int32, (rb, n), 1)
    neg = jnp.array(-jnp.inf, _F32)
    scores = jnp.where(cols > rows, neg, scores)
    keep = jnp.zeros((rb, n), jnp.bool_)
    for _ in range(_K):
        cur = jnp.where(keep, neg, scores)
        m = jnp.max(cur, axis=1, keepdims=True)
        jstar = jnp.min(jnp.where(cur == m, cols, n), axis=1, keepdims=True)
        keep = jnp.logical_or(keep, cols == jstar)
    masked = jnp.where(keep, scores, neg)
    mrow = jnp.max(masked, axis=1, keepdims=True)
    p = jnp.exp(masked - mrow)
    attn = p / jnp.sum(p, axis=1, keepdims=True)
    g = jax.lax.dot_general(attn, v_ref[0], (((1,), (0,)), ((), ())),
                            preferred_element_type=_F32)
    out_ref[0] = part_ref[0] + jax.lax.dot_general(
        g, wobt_ref[...], (((1,), (0,)), ((), ())),
        preferred_element_type=_F32)


def kernel(mu, pheromone, Wq, bq, Wk, bk, Wv, bv, W1, b1, W2, b2, Wo, bo):
    b_, n, d = mu.shape
    rows = b_ * n

    w1a_t = W1[:, :d].T
    w1b_t = W1[:, d:].T
    wcat = jnp.concatenate([Wq.T, Wk.T, Wv.T, w1a_t, w1b_t], axis=1)
    bcat = jnp.concatenate(
        [bq, bk, bv, jnp.zeros((2 * d,), _F32)])[None, :]
    w2t = W2.T
    woa_t = Wo[:, :d].T
    wob_t = Wo[:, d:].T
    bvec = (b2[None, :] @ woa_t + bo[None, :])
    b1row = b1[None, :]

    rb_a = min(512, rows)
    mu_flat = mu.reshape(rows, d)
    outs = pl.pallas_call(
        _proj_kernel,
        grid=(rows // rb_a,),
        in_specs=[
            pl.BlockSpec((rb_a, d), lambda i: (i, 0)),
            pl.BlockSpec((d, 5 * d), lambda i: (0, 0)),
            pl.BlockSpec((1, 5 * d), lambda i: (0, 0)),
        ],
        out_specs=[pl.BlockSpec((rb_a, d), lambda i: (i, 0))] * 5,
        out_shape=[jax.ShapeDtypeStruct((rows, d), _F32)] * 5,
    )(mu_flat, wcat, bcat)
    q, k, v, s, t = (o.reshape(b_, n, d) for o in outs)

    rb_b = min(512, n)
    nb = n // rb_b
    part = pl.pallas_call(
        _local_kernel,
        grid=(b_, nb),
        in_specs=[
            pl.BlockSpec((1, rb_b, d), lambda b, i: (b, i, 0)),
            pl.BlockSpec((1, rb_b, d), lambda b, i: (b, i, 0)),
            pl.BlockSpec((1, rb_b, d),
                         lambda b, i: (b, jnp.maximum(i - 1, 0), 0)),
            pl.BlockSpec((d, d), lambda b, i: (0, 0)),
            pl.BlockSpec((d, d), lambda b, i: (0, 0)),
            pl.BlockSpec((1, d), lambda b, i: (0, 0)),
            pl.BlockSpec((1, d), lambda b, i: (0, 0)),
        ],
        out_specs=pl.BlockSpec((1, rb_b, d), lambda b, i: (b, i, 0)),
        out_shape=jax.ShapeDtypeStruct((b_, n, d), _F32),
    )(s, t, t, w2t, woa_t, b1row, bvec)

    rb_c = min(256, n)
    nc = n // rb_c
    ph3 = pheromone[:, None, :]
    out = pl.pallas_call(
        _attn_kernel,
        grid=(b_, nc),
        in_specs=[
            pl.BlockSpec((1, rb_c, d), lambda b, i: (b, i, 0)),
            pl.BlockSpec((1, n, d), lambda b, i: (b, 0, 0)),
            pl.BlockSpec((1, n, d), lambda b, i: (b, 0, 0)),
            pl.BlockSpec((1, 1, n), lambda b, i: (b, 0, 0)),
            pl.BlockSpec((1, rb_c, d), lambda b, i: (b, i, 0)),
            pl.BlockSpec((d, d), lambda b, i: (0, 0)),
        ],
        out_specs=pl.BlockSpec((1, rb_c, d), lambda b, i: (b, i, 0)),
        out_shape=jax.ShapeDtypeStruct((b_, n, d), _F32),
    )(q, k, v, ph3, part, wob_t)
    return out


# speedup vs baseline: 11.4782x; 1.6107x over previous
import functools
import math

import jax
import jax.numpy as jnp
from jax.experimental import pallas as pl

_WINDOW, _K, _ALPHA = 4, 8, 0.3
_F32 = jnp.float32
_BF16 = jnp.bfloat16


def _projlocal_kernel(mu_ref, mup_ref, wqk_ref, bqk_ref, wvst_ref, w2o_ref,
                      b1_ref, bvec_ref, q_ref, k_ref, v_ref, part_ref):
    i = pl.program_id(1)
    rb, d = mu_ref.shape[1], mu_ref.shape[2]
    mu = mu_ref[0]
    qk = jax.lax.dot_general(mu, wqk_ref[...], (((1,), (0,)), ((), ())),
                             preferred_element_type=_F32) + bqk_ref[...]
    q_ref[0] = qk[:, :d]
    k_ref[0] = qk[:, d:]
    mu_bf = mu.astype(_BF16)
    vst = jax.lax.dot_general(mu_bf, wvst_ref[...], (((1,), (0,)), ((), ())),
                              preferred_element_type=_F32)
    v_ref[0] = vst[:, :d].astype(_BF16)
    s = vst[:, d:2 * d]
    t = vst[:, 2 * d:]
    prev_bf = mup_ref[0][rb - 8:, :].astype(_BF16)
    t_tail = jax.lax.dot_general(prev_bf, wvst_ref[:, 2 * d:],
                                 (((1,), (0,)), ((), ())),
                                 preferred_element_type=_F32)
    live = (i > 0).astype(_F32)
    tail = t_tail[8 - _WINDOW:, :] * live
    text = jnp.concatenate([tail, t], axis=0)
    b1 = b1_ref[...]
    acc = jnp.zeros_like(s)
    for w in range(1, _WINDOW + 1):
        x = s + text[_WINDOW - w:_WINDOW - w + rb, :] + b1
        acc = acc + x * jax.nn.sigmoid(x)
    h = (acc * (1.0 / _WINDOW)).astype(_BF16)
    part_ref[0] = jax.lax.dot_general(
        h, w2o_ref[...], (((1,), (0,)), ((), ())),
        preferred_element_type=_F32) + bvec_ref[...]


def _attn_kernel(q_ref, k_ref, v_ref, ph_ref, part_ref, out_ref):
    i = pl.program_id(1)
    rb = q_ref.shape[1]
    n = k_ref.shape[1]
    s0 = jax.lax.dot_general(q_ref[0], k_ref[0], (((1,), (1,)), ((), ())),
                             preferred_element_type=_F32)
    rows = i * rb + jax.lax.broadcasted_iota(jnp.int32, (rb, n), 0)
    cols = jax.lax.broadcasted_iota(jnp.int32, (rb, n), 1)
    neg = jnp.array(-jnp.inf, _F32)
    scores = jnp.where(cols > rows, neg, s0 + ph_ref[0])
    m1 = jnp.max(scores, axis=1, keepdims=True)
    c = scores
    m = m1
    for _ in range(_K - 1):
        c = jnp.where(c >= m, neg, c)
        m = jnp.max(c, axis=1, keepdims=True)
    p = jnp.where(scores >= m, jnp.exp(scores - m1), 0.0)
    z = jnp.sum(p, axis=1, keepdims=True)
    g = jax.lax.dot_general(p.astype(_BF16), v_ref[0],
                            (((1,), (0,)), ((), ())),
                            preferred_element_type=_F32)
    out_ref[0] = part_ref[0] + g * (1.0 / z)


def kernel(mu, pheromone, Wq, bq, Wk, bk, Wv, bv, W1, b1, W2, b2, Wo, bo):
    b_, n, d = mu.shape

    inv_sqrt = 1.0 / math.sqrt(d)
    wqk = jnp.concatenate([Wq.T * inv_sqrt, Wk.T], axis=1)
    bqk = jnp.concatenate([bq * inv_sqrt, bk])[None, :]
    woa_t = Wo[:, :d].T
    wob_t = Wo[:, d:].T
    wv_fold = Wv.T @ wob_t
    wvst = jnp.concatenate([wv_fold, W1[:, :d].T, W1[:, d:].T],
                           axis=1).astype(_BF16)
    w2o = (W2.T @ woa_t).astype(_BF16)
    bvec = (b2[None, :] @ woa_t + bo[None, :] + bv[None, :] @ wob_t)
    b1row = b1[None, :]
    ph3 = (_ALPHA * pheromone)[:, None, :]

    rb = min(512, n)
    nb = n // rb
    q, k, v, part = pl.pallas_call(
        _projlocal_kernel,
        grid=(b_, nb),
        in_specs=[
            pl.BlockSpec((1, rb, d), lambda b, i: (b, i, 0)),
            pl.BlockSpec((1, rb, d), lambda b, i: (b, jnp.maximum(i - 1, 0), 0)),
            pl.BlockSpec((d, 2 * d), lambda b, i: (0, 0)),
            pl.BlockSpec((1, 2 * d), lambda b, i: (0, 0)),
            pl.BlockSpec((d, 3 * d), lambda b, i: (0, 0)),
            pl.BlockSpec((d, d), lambda b, i: (0, 0)),
            pl.BlockSpec((1, d), lambda b, i: (0, 0)),
            pl.BlockSpec((1, d), lambda b, i: (0, 0)),
        ],
        out_specs=[pl.BlockSpec((1, rb, d), lambda b, i: (b, i, 0))] * 4,
        out_shape=[
            jax.ShapeDtypeStruct((b_, n, d), _F32),
            jax.ShapeDtypeStruct((b_, n, d), _F32),
            jax.ShapeDtypeStruct((b_, n, d), _BF16),
            jax.ShapeDtypeStruct((b_, n, d), _F32),
        ],
    )(mu, mu, wqk, bqk, wvst, w2o, b1row, bvec)

    rb_c = min(256, n)
    nc = n // rb_c
    out = pl.pallas_call(
        _attn_kernel,
        grid=(b_, nc),
        in_specs=[
            pl.BlockSpec((1, rb_c, d), lambda b, i: (b, i, 0)),
            pl.BlockSpec((1, n, d), lambda b, i: (b, 0, 0)),
            pl.BlockSpec((1, n, d), lambda b, i: (b, 0, 0)),
            pl.BlockSpec((1, 1, n), lambda b, i: (b, 0, 0)),
            pl.BlockSpec((1, rb_c, d), lambda b, i: (b, i, 0)),
        ],
        out_specs=pl.BlockSpec((1, rb_c, d), lambda b, i: (b, i, 0)),
        out_shape=jax.ShapeDtypeStruct((b_, n, d), _F32),
    )(q, k, v, ph3, part)
    return out


# speedup vs baseline: 12.4705x; 1.0865x over previous
import functools
import math

import jax
import jax.numpy as jnp
from jax.experimental import pallas as pl

_WINDOW, _K, _ALPHA = 4, 8, 0.3
_F32 = jnp.float32
_BF16 = jnp.bfloat16

_NT = (((1,), (1,)), ((), ()))
_NN = (((1,), (0,)), ((), ()))


def _fold_kernel(wo_ref, w2_ref, wv_ref, w1_ref, b2_ref, bv_ref, bo_ref,
                 w3_ref, fl_ref, bvec_ref):
    d = w2_ref.shape[0]
    woa = wo_ref[:, :d]
    wob = wo_ref[:, d:]
    w3_ref[:d] = jax.lax.dot_general(
        wob, wv_ref[...], _NN, preferred_element_type=_F32).astype(_BF16)
    w3_ref[d:2 * d] = w1_ref[:, :d].astype(_BF16)
    w3_ref[2 * d:] = w1_ref[:, d:].astype(_BF16)
    fl_ref[...] = jax.lax.dot_general(
        woa, w2_ref[...], _NN, preferred_element_type=_F32).astype(_BF16)
    bvec_ref[...] = (
        jax.lax.dot_general(b2_ref[...], woa, _NT,
                            preferred_element_type=_F32) +
        jax.lax.dot_general(bv_ref[...], wob, _NT,
                            preferred_element_type=_F32) + bo_ref[...])


def _projlocal_kernel(mu_ref, mup_ref, wq_ref, wk_ref, bq_ref, bk_ref,
                      w3_ref, fl_ref, b1_ref, bvec_ref,
                      q_ref, k_ref, v_ref, part_ref):
    i = pl.program_id(1)
    rb, d = mu_ref.shape[1], mu_ref.shape[2]
    inv_sqrt = 1.0 / math.sqrt(d)
    mu = mu_ref[0]
    q_ref[0] = (jax.lax.dot_general(mu, wq_ref[...], _NT,
                                    preferred_element_type=_F32)
                + bq_ref[...]) * inv_sqrt
    k_ref[0] = jax.lax.dot_general(mu, wk_ref[...], _NT,
                                   preferred_element_type=_F32) + bk_ref[...]
    mu_bf = mu.astype(_BF16)
    vst = jax.lax.dot_general(mu_bf, w3_ref[...], _NT,
                              preferred_element_type=_F32)
    v_ref[0] = vst[:, :d].astype(_BF16)
    s = vst[:, d:2 * d]
    t = vst[:, 2 * d:]
    prev_bf = mup_ref[0][rb - 8:, :].astype(_BF16)
    t_tail = jax.lax.dot_general(prev_bf, w3_ref[2 * d:, :], _NT,
                                 preferred_element_type=_F32)
    live = (i > 0).astype(_F32)
    tail = t_tail[8 - _WINDOW:, :] * live
    text = jnp.concatenate([tail, t], axis=0)
    b1 = b1_ref[...]
    acc = jnp.zeros_like(s)
    for w in range(1, _WINDOW + 1):
        x = s + text[_WINDOW - w:_WINDOW - w + rb, :] + b1
        acc = acc + x * jax.nn.sigmoid(x)
    h = (acc * (1.0 / _WINDOW)).astype(_BF16)
    part_ref[0] = jax.lax.dot_general(
        h, fl_ref[...], _NT, preferred_element_type=_F32) + bvec_ref[...]


def _attn_kernel(q_ref, k_ref, v_ref, ph_ref, part_ref, out_ref):
    i = pl.program_id(1)
    rb = q_ref.shape[1]
    n = k_ref.shape[1]
    s0 = jax.lax.dot_general(q_ref[0], k_ref[0], _NT,
                             preferred_element_type=_F32)
    rows = i * rb + jax.lax.broadcasted_iota(jnp.int32, (rb, n), 0)
    cols = jax.lax.broadcasted_iota(jnp.int32, (rb, n), 1)
    neg = jnp.array(-jnp.inf, _F32)
    scores = jnp.where(cols > rows, neg, s0 + _ALPHA * ph_ref[0])
    m1 = jnp.max(scores, axis=1, keepdims=True)
    c = scores
    m = m1
    for _ in range(_K - 1):
        c = jnp.where(c >= m, neg, c)
        m = jnp.max(c, axis=1, keepdims=True)
    p = jnp.where(scores >= m, jnp.exp(scores - m1), 0.0)
    z = jnp.sum(p, axis=1, keepdims=True)
    g = jax.lax.dot_general(p.astype(_BF16), v_ref[0], _NN,
                            preferred_element_type=_F32)
    out_ref[0] = part_ref[0] + g * (1.0 / z)


def kernel(mu, pheromone, Wq, bq, Wk, bk, Wv, bv, W1, b1, W2, b2, Wo, bo):
    b_, n, d = mu.shape

    w3, fl, bvec = pl.pallas_call(
        _fold_kernel,
        out_shape=[
            jax.ShapeDtypeStruct((3 * d, d), _BF16),
            jax.ShapeDtypeStruct((d, d), _BF16),
            jax.ShapeDtypeStruct((1, d), _F32),
        ],
    )(Wo, W2, Wv, W1, b2[None, :], bv[None, :], bo[None, :])

    rb = min(512, n)
    nb = n // rb
    q, k, v, part = pl.pallas_call(
        _projlocal_kernel,
        grid=(b_, nb),
        in_specs=[
            pl.BlockSpec((1, rb, d), lambda b, i: (b, i, 0)),
            pl.BlockSpec((1, rb, d), lambda b, i: (b, jnp.maximum(i - 1, 0), 0)),
            pl.BlockSpec((d, d), lambda b, i: (0, 0)),
            pl.BlockSpec((d, d), lambda b, i: (0, 0)),
            pl.BlockSpec((1, d), lambda b, i: (0, 0)),
            pl.BlockSpec((1, d), lambda b, i: (0, 0)),
            pl.BlockSpec((3 * d, d), lambda b, i: (0, 0)),
            pl.BlockSpec((d, d), lambda b, i: (0, 0)),
            pl.BlockSpec((1, d), lambda b, i: (0, 0)),
            pl.BlockSpec((1, d), lambda b, i: (0, 0)),
        ],
        out_specs=[pl.BlockSpec((1, rb, d), lambda b, i: (b, i, 0))] * 4,
        out_shape=[
            jax.ShapeDtypeStruct((b_, n, d), _F32),
            jax.ShapeDtypeStruct((b_, n, d), _F32),
            jax.ShapeDtypeStruct((b_, n, d), _BF16),
            jax.ShapeDtypeStruct((b_, n, d), _F32),
        ],
    )(mu, mu, Wq, Wk, bq[None, :], bk[None, :], w3, fl, b1[None, :], bvec)

    rb_c = min(256, n)
    nc = n // rb_c
    out = pl.pallas_call(
        _attn_kernel,
        grid=(b_, nc),
        in_specs=[
            pl.BlockSpec((1, rb_c, d), lambda b, i: (b, i, 0)),
            pl.BlockSpec((1, n, d), lambda b, i: (b, 0, 0)),
            pl.BlockSpec((1, n, d), lambda b, i: (b, 0, 0)),
            pl.BlockSpec((1, 1, n), lambda b, i: (b, 0, 0)),
            pl.BlockSpec((1, rb_c, d), lambda b, i: (b, i, 0)),
        ],
        out_specs=pl.BlockSpec((1, rb_c, d), lambda b, i: (b, i, 0)),
        out_shape=jax.ShapeDtypeStruct((b_, n, d), _F32),
    )(q, k, v, pheromone[:, None, :], part)
    return out


# speedup vs baseline: 12.9130x; 1.0355x over previous
import functools
import math

import jax
import jax.numpy as jnp
from jax.experimental import pallas as pl

_WINDOW, _K, _ALPHA = 4, 8, 0.3
_F32 = jnp.float32
_BF16 = jnp.bfloat16

_NT = (((1,), (1,)), ((), ()))
_NN = (((1,), (0,)), ((), ()))


def _fold_kernel(wo_ref, w2_ref, wv_ref, w1_ref, b2_ref, bv_ref, bo_ref,
                 w3_ref, fl_ref, bvec_ref):
    d = w2_ref.shape[0]
    woa = wo_ref[:, :d]
    wob = wo_ref[:, d:]
    w3_ref[:d] = jax.lax.dot_general(
        wob, wv_ref[...], _NN, preferred_element_type=_F32).astype(_BF16)
    w3_ref[d:2 * d] = w1_ref[:, :d].astype(_BF16)
    w3_ref[2 * d:] = w1_ref[:, d:].astype(_BF16)
    fl_ref[...] = jax.lax.dot_general(
        woa, w2_ref[...], _NN, preferred_element_type=_F32).astype(_BF16)
    bvec_ref[...] = (
        jax.lax.dot_general(b2_ref[...], woa, _NT,
                            preferred_element_type=_F32) +
        jax.lax.dot_general(bv_ref[...], wob, _NT,
                            preferred_element_type=_F32) + bo_ref[...])


def _projlocal_kernel(mu_ref, mup_ref, wq_ref, wk_ref, bq_ref, bk_ref,
                      w3_ref, fl_ref, b1_ref, bvec_ref,
                      q_ref, k_ref, v_ref, part_ref):
    i = pl.program_id(1)
    rb, d = mu_ref.shape[1], mu_ref.shape[2]
    inv_sqrt = 1.0 / math.sqrt(d)
    mu = mu_ref[0]
    q_ref[0] = (jax.lax.dot_general(mu, wq_ref[...], _NT,
                                    preferred_element_type=_F32)
                + bq_ref[...]) * inv_sqrt
    k_ref[0] = jax.lax.dot_general(mu, wk_ref[...], _NT,
                                   preferred_element_type=_F32) + bk_ref[...]
    mu_bf = mu.astype(_BF16)
    vst = jax.lax.dot_general(mu_bf, w3_ref[...], _NT,
                              preferred_element_type=_F32)
    v_ref[0] = vst[:, :d].astype(_BF16)
    s = vst[:, d:2 * d]
    t = vst[:, 2 * d:]
    prev_bf = mup_ref[0][rb - 8:, :].astype(_BF16)
    t_tail = jax.lax.dot_general(prev_bf, w3_ref[2 * d:, :], _NT,
                                 preferred_element_type=_F32)
    live = (i > 0).astype(_F32)
    tail = t_tail[8 - _WINDOW:, :] * live
    text = jnp.concatenate([tail, t], axis=0)
    b1 = b1_ref[...]
    acc = jnp.zeros_like(s)
    for w in range(1, _WINDOW + 1):
        x = s + text[_WINDOW - w:_WINDOW - w + rb, :] + b1
        acc = acc + x * jax.nn.sigmoid(x)
    h = (acc * (1.0 / _WINDOW)).astype(_BF16)
    part_ref[0] = jax.lax.dot_general(
        h, fl_ref[...], _NT, preferred_element_type=_F32) + bvec_ref[...]


def _attn_kernel(q_ref, k_ref, v_ref, ph_ref, part_ref, out_ref):
    i = pl.program_id(1)
    rb = q_ref.shape[1]
    n = k_ref.shape[1]
    s0 = jax.lax.dot_general(q_ref[0], k_ref[0], _NT,
                             preferred_element_type=_F32)
    rows = i * rb + jax.lax.broadcasted_iota(jnp.int32, (rb, n), 0)
    cols = jax.lax.broadcasted_iota(jnp.int32, (rb, n), 1)
    neg = jnp.array(-jnp.inf, _F32)
    scores = jnp.where(cols > rows, neg, s0 + _ALPHA * ph_ref[0])
    m1 = jnp.max(scores, axis=1, keepdims=True)
    c = scores
    m = m1
    for _ in range(_K - 1):
        c = jnp.where(c >= m, neg, c)
        m = jnp.max(c, axis=1, keepdims=True)
    p = jnp.where(scores >= m, jnp.exp(scores - m1), 0.0)
    z = jnp.sum(p, axis=1, keepdims=True)
    g = jax.lax.dot_general(p.astype(_BF16), v_ref[0], _NN,
                            preferred_element_type=_F32)
    out_ref[0] = part_ref[0] + g * (1.0 / z)


def kernel(mu, pheromone, Wq, bq, Wk, bk, Wv, bv, W1, b1, W2, b2, Wo, bo):
    b_, n, d = mu.shape

    w3, fl, bvec = pl.pallas_call(
        _fold_kernel,
        out_shape=[
            jax.ShapeDtypeStruct((3 * d, d), _BF16),
            jax.ShapeDtypeStruct((d, d), _BF16),
            jax.ShapeDtypeStruct((1, d), _F32),
        ],
    )(Wo, W2, Wv, W1, b2[None, :], bv[None, :], bo[None, :])

    rb = min(512, n)
    nb = n // rb
    q, k, v, part = pl.pallas_call(
        _projlocal_kernel,
        grid=(b_, nb),
        in_specs=[
            pl.BlockSpec((1, rb, d), lambda b, i: (b, i, 0)),
            pl.BlockSpec((1, rb, d), lambda b, i: (b, jnp.maximum(i - 1, 0), 0)),
            pl.BlockSpec((d, d), lambda b, i: (0, 0)),
            pl.BlockSpec((d, d), lambda b, i: (0, 0)),
            pl.BlockSpec((1, d), lambda b, i: (0, 0)),
            pl.BlockSpec((1, d), lambda b, i: (0, 0)),
            pl.BlockSpec((3 * d, d), lambda b, i: (0, 0)),
            pl.BlockSpec((d, d), lambda b, i: (0, 0)),
            pl.BlockSpec((1, d), lambda b, i: (0, 0)),
            pl.BlockSpec((1, d), lambda b, i: (0, 0)),
        ],
        out_specs=[pl.BlockSpec((1, rb, d), lambda b, i: (b, i, 0))] * 4,
        out_shape=[
            jax.ShapeDtypeStruct((b_, n, d), _F32),
            jax.ShapeDtypeStruct((b_, n, d), _F32),
            jax.ShapeDtypeStruct((b_, n, d), _BF16),
            jax.ShapeDtypeStruct((b_, n, d), _F32),
        ],
    )(mu, mu, Wq, Wk, bq[None, :], bk[None, :], w3, fl, b1[None, :], bvec)

    rb_c = min(512, n)
    nc = n // rb_c
    out = pl.pallas_call(
        _attn_kernel,
        grid=(b_, nc),
        in_specs=[
            pl.BlockSpec((1, rb_c, d), lambda b, i: (b, i, 0)),
            pl.BlockSpec((1, n, d), lambda b, i: (b, 0, 0)),
            pl.BlockSpec((1, n, d), lambda b, i: (b, 0, 0)),
            pl.BlockSpec((1, 1, n), lambda b, i: (b, 0, 0)),
            pl.BlockSpec((1, rb_c, d), lambda b, i: (b, i, 0)),
        ],
        out_specs=pl.BlockSpec((1, rb_c, d), lambda b, i: (b, i, 0)),
        out_shape=jax.ShapeDtypeStruct((b_, n, d), _F32),
    )(q, k, v, pheromone[:, None, :], part)
    return out
